# baseline (speedup 1.0000x reference)
import functools

import jax
import jax.numpy as jnp
from jax import lax
from jax.experimental import pallas as pl
from jax.experimental.pallas import tpu as pltpu
from jax.experimental.pallas import tpu_sc as plsc

F = 2048
D = 64
E = 65504
NC = 2
NS = 16
NW = NC * NS
EW = 2048
CH = 128
NCH = EW // CH
VROWS = F + 128
DD = 2 * D
RPT = VROWS // NS
TAIL = NW * EW - E
REAL_TAIL = CH - TAIL


def _sc_segment_sum(w_cat, src_ids, dst_ids, zeros):
    mesh = plsc.VectorSubcoreMesh(core_axis_name="c", subcore_axis_name="s")

    @functools.partial(
        pl.kernel,
        out_type=jax.ShapeDtypeStruct((NC * VROWS, DD), jnp.float32),
        mesh=mesh,
        scratch_types=[
            pltpu.VMEM((EW,), jnp.int32),
            pltpu.VMEM((EW,), jnp.int32),
            pltpu.VMEM((CH, DD), jnp.float32),
            pltpu.VMEM((CH, DD), jnp.float32),
            pltpu.VMEM((CH, DD), jnp.float32),
            pltpu.VMEM((CH, DD), jnp.float32),
            pltpu.VMEM_SHARED((VROWS, DD), jnp.float32),
            pltpu.SemaphoreType.DMA,
            pltpu.SemaphoreType.DMA,
            pltpu.SemaphoreType.DMA,
            pltpu.SemaphoreType.DMA,
            pltpu.SemaphoreType.DMA,
        ],
    )
    def seg(w_hbm, s_hbm, d_hbm, z_hbm, out_hbm,
            src_v, dst_v, buf0, buf1, buf2, buf3, v_sh,
            sem0, sem1, sem2, sem3, semi):
        c = lax.axis_index("c")
        s = lax.axis_index("s")
        wid = c * NS + s
        base = wid * EW

        zcp = pltpu.make_async_copy(z_hbm, v_sh.at[pl.ds(s * RPT, RPT)], semi)
        zcp.start()

        last = wid == NW - 1

        @pl.when(jnp.logical_not(last))
        def _():
            pltpu.sync_copy(s_hbm.at[pl.ds(base, EW)], src_v)
            pltpu.sync_copy(d_hbm.at[pl.ds(base, EW)], dst_v)

        @pl.when(last)
        def _():
            pltpu.sync_copy(s_hbm.at[pl.ds(base, EW - TAIL)],
                            src_v.at[pl.ds(0, EW - TAIL)])
            pltpu.sync_copy(d_hbm.at[pl.ds(base, EW - TAIL)],
                            dst_v.at[pl.ds(0, EW - TAIL)])
            for t in range(EW - TAIL, EW, 16):
                src_v[pl.ds(t, 16)] = jnp.zeros((16,), jnp.int32)
                dst_v[pl.ds(t, 16)] = jnp.full((16,), F, jnp.int32)

        zcp.wait()
        plsc.subcore_barrier()

        NB = 4
        bufs = (buf0, buf1, buf2, buf3)
        sems = (sem0, sem1, sem2, sem3)

        def gcp(j):
            return pltpu.make_async_copy(
                w_hbm.at[src_v.at[pl.ds(j * CH, CH)]],
                bufs[j % NB], sems[j % NB])

        for j in range(NB):
            gcp(j).start()
        for j in range(NCH):
            gcp(j).wait()
            pltpu.sync_copy(bufs[j % NB],
                            v_sh.at[dst_v.at[pl.ds(j * CH, CH)]], add=True)
            if j + NB < NCH:
                gcp(j + NB).start()

        plsc.subcore_barrier()
        pltpu.sync_copy(v_sh.at[pl.ds(s * RPT, RPT)],
                        out_hbm.at[pl.ds(c * VROWS + s * RPT, RPT)])

    return seg(w_cat, src_ids, dst_ids, zeros)


def _tc_body(vpart_hbm, wout0_ref, wout1_ref, femb_ref,
             wrz_ref, wn_ref, un_ref, brz_ref, bn_ref, cn_ref,
             b0_ref, b1_ref, out_ref, vbuf, vsem):
    cp0 = pltpu.make_async_copy(vpart_hbm.at[pl.ds(0, F)], vbuf.at[0], vsem)
    cp1 = pltpu.make_async_copy(vpart_hbm.at[pl.ds(VROWS, F)], vbuf.at[1],
                                vsem)
    cp0.start()
    cp1.start()
    wcat = jnp.concatenate([wout0_ref[...], wout1_ref[...]], axis=1)
    cp0.wait()
    cp1.wait()
    vsum = vbuf[0] + vbuf[1]
    dn = (((0,), (0,)), ((), ()))
    x = lax.dot_general(wcat, vsum, dn,
                        preferred_element_type=jnp.float32)
    m0 = x[:D, :D]
    m1 = x[D:, D:]
    femb = femb_ref[...]
    wrz = wrz_ref[...]
    wn, un = wn_ref[...], un_ref[...]
    brz = brz_ref[...]

    def mm(a, w):
        return lax.dot_general(a, w, (((1,), (0,)), ((), ())),
                               preferred_element_type=jnp.float32)

    h = femb
    for m, b_ref in ((m0, b0_ref), (m1, b1_ref)):
        a = jnp.dot(h, m, preferred_element_type=jnp.float32) + b_ref[...]
        ah = jnp.concatenate([a, h], axis=1)
        rz = jax.nn.sigmoid(mm(ah, wrz) + brz)
        r = rz[:, :D]
        z = rz[:, D:]
        n = jnp.tanh(mm(a, wn) + bn_ref[...] + r * (mm(h, un) + cn_ref[...]))
        h = (1.0 - z) * n + z * h + femb
    out_ref[...] = h


def kernel(feature_emb, edge_index, W_out0, W_in0, bias0,
           W_out1, W_in1, bias1, W_ih, W_hh, b_ih, b_hh):
    w_cat = jnp.concatenate([W_in0, W_in1], axis=1)
    src_ids = edge_index[0].astype(jnp.int32)
    dst_ids = edge_index[1].astype(jnp.int32)
    zeros = jnp.zeros((RPT, DD), jnp.float32)

    vpart = _sc_segment_sum(w_cat, src_ids, dst_ids, zeros)

    wrz = jnp.concatenate(
        [jnp.concatenate([W_ih[:D].T, W_ih[D:2 * D].T], axis=1),
         jnp.concatenate([W_hh[:D].T, W_hh[D:2 * D].T], axis=1)], axis=0)
    brz = (jnp.concatenate([b_ih[:D] + b_hh[:D],
                            b_ih[D:2 * D] + b_hh[D:2 * D]])).reshape(1, 2 * D)
    wn = W_ih[2 * D:].T
    un = W_hh[2 * D:].T
    bn = b_ih[2 * D:].reshape(1, D)
    cn = b_hh[2 * D:].reshape(1, D)

    return pl.pallas_call(
        _tc_body,
        out_shape=jax.ShapeDtypeStruct((F, D), jnp.float32),
        in_specs=[pl.BlockSpec(memory_space=pltpu.MemorySpace.HBM)]
        + [pl.BlockSpec(memory_space=pltpu.MemorySpace.VMEM)] * 11,
        scratch_shapes=[pltpu.VMEM((2, F, DD), jnp.float32),
                        pltpu.SemaphoreType.DMA],
    )(vpart, W_out0, W_out1, feature_emb,
      wrz, wn, un, brz, bn, cn,
      bias0.reshape(1, D), bias1.reshape(1, D))


# speedup vs baseline: 1.0222x; 1.0222x over previous
import functools

import jax
import jax.numpy as jnp
from jax import lax
from jax.experimental import pallas as pl
from jax.experimental.pallas import tpu as pltpu
from jax.experimental.pallas import tpu_sc as plsc

F = 2048
D = 64
E = 65504
NC = 2
NS = 16
NW = NC * NS
EW = 2048
CH = 128
NCH = EW // CH
VROWS = F + 128
DD = 2 * D
RPT = VROWS // NS
TAIL = NW * EW - E
REAL_TAIL = CH - TAIL


def _sc_segment_sum(w_cat, eidx_flat, zeros):
    mesh = plsc.VectorSubcoreMesh(core_axis_name="c", subcore_axis_name="s")

    @functools.partial(
        pl.kernel,
        out_type=jax.ShapeDtypeStruct((NC * VROWS, DD), jnp.float32),
        mesh=mesh,
        scratch_types=[
            pltpu.VMEM((EW,), jnp.int32),
            pltpu.VMEM((EW,), jnp.int32),
            [pltpu.VMEM((CH, DD), jnp.float32)] * 6,
            pltpu.VMEM_SHARED((VROWS, DD), jnp.float32),
            [pltpu.SemaphoreType.DMA] * 6,
            [pltpu.SemaphoreType.DMA] * 6,
            pltpu.SemaphoreType.DMA,
        ],
    )
    def seg(w_hbm, e_hbm, z_hbm, out_hbm,
            src_v, dst_v, bufs, v_sh, gsems, ssems, semi):
        c = lax.axis_index("c")
        s = lax.axis_index("s")
        wid = c * NS + s
        base = wid * EW

        zcp = pltpu.make_async_copy(z_hbm, v_sh.at[pl.ds(s * RPT, RPT)], semi)
        zcp.start()

        last = wid == NW - 1

        @pl.when(jnp.logical_not(last))
        def _():
            pltpu.sync_copy(e_hbm.at[pl.ds(base, EW)], src_v)
            pltpu.sync_copy(e_hbm.at[pl.ds(E + base, EW)], dst_v)

        @pl.when(last)
        def _():
            pltpu.sync_copy(e_hbm.at[pl.ds(base, EW - TAIL)],
                            src_v.at[pl.ds(0, EW - TAIL)])
            pltpu.sync_copy(e_hbm.at[pl.ds(E + base, EW - TAIL)],
                            dst_v.at[pl.ds(0, EW - TAIL)])
            for t in range(EW - TAIL, EW, 16):
                src_v[pl.ds(t, 16)] = jnp.zeros((16,), jnp.int32)
                dst_v[pl.ds(t, 16)] = jnp.full((16,), F, jnp.int32)

        zcp.wait()
        plsc.subcore_barrier()

        NB = 6

        def gcp(j):
            return pltpu.make_async_copy(
                w_hbm.at[src_v.at[pl.ds(j * CH, CH)]],
                bufs[j % NB], gsems[j % NB])

        sstarted = {}

        def scp_start(j):
            sstarted[j] = pltpu.async_copy(
                bufs[j % NB], v_sh.at[dst_v.at[pl.ds(j * CH, CH)]],
                ssems[j % NB], add=True)

        DEPTH = 4
        for j in range(DEPTH):
            gcp(j).start()
        for j in range(NCH):
            gcp(j).wait()
            scp_start(j)
            n = j + DEPTH
            if n < NCH:
                if n - NB >= 0:
                    sstarted[n - NB].wait()
                gcp(n).start()
        for j in range(max(0, NCH - NB), NCH):
            sstarted[j].wait()

        plsc.subcore_barrier()
        pltpu.sync_copy(v_sh.at[pl.ds(s * RPT, RPT)],
                        out_hbm.at[pl.ds(c * VROWS + s * RPT, RPT)])

    return seg(w_cat, eidx_flat, zeros)


def _tc_body(vpart_ref, wout0_ref, wout1_ref, femb_ref,
             wrz_ref, wn_ref, un_ref, brz_ref, bn_ref, cn_ref,
             b0_ref, b1_ref, out_ref):
    vsum = vpart_ref[:F, :] + vpart_ref[VROWS:VROWS + F, :]
    wcat = jnp.concatenate([wout0_ref[...], wout1_ref[...]], axis=1)
    dn = (((0,), (0,)), ((), ()))
    x = lax.dot_general(wcat, vsum, dn,
                        preferred_element_type=jnp.float32)
    m0 = x[:D, :D]
    m1 = x[D:, D:]
    femb = femb_ref[...]
    wrz = wrz_ref[...]
    wn, un = wn_ref[...], un_ref[...]
    brz = brz_ref[...]

    def mm(a, w):
        return lax.dot_general(a, w, (((1,), (0,)), ((), ())),
                               preferred_element_type=jnp.float32)

    h = femb
    for m, b_ref in ((m0, b0_ref), (m1, b1_ref)):
        a = jnp.dot(h, m, preferred_element_type=jnp.float32) + b_ref[...]
        ah = jnp.concatenate([a, h], axis=1)
        rz = jax.nn.sigmoid(mm(ah, wrz) + brz)
        r = rz[:, :D]
        z = rz[:, D:]
        n = jnp.tanh(mm(a, wn) + bn_ref[...] + r * (mm(h, un) + cn_ref[...]))
        h = (1.0 - z) * n + z * h + femb
    out_ref[...] = h


def kernel(feature_emb, edge_index, W_out0, W_in0, bias0,
           W_out1, W_in1, bias1, W_ih, W_hh, b_ih, b_hh):
    w_cat = jnp.concatenate([W_in0, W_in1], axis=1)
    eidx_flat = edge_index.astype(jnp.int32).reshape(-1)
    zeros = jnp.zeros((RPT, DD), jnp.float32)

    vpart = _sc_segment_sum(w_cat, eidx_flat, zeros)

    wrz = jnp.concatenate(
        [jnp.concatenate([W_ih[:D].T, W_ih[D:2 * D].T], axis=1),
         jnp.concatenate([W_hh[:D].T, W_hh[D:2 * D].T], axis=1)], axis=0)
    brz = (jnp.concatenate([b_ih[:D] + b_hh[:D],
                            b_ih[D:2 * D] + b_hh[D:2 * D]])).reshape(1, 2 * D)
    wn = W_ih[2 * D:].T
    un = W_hh[2 * D:].T
    bn = b_ih[2 * D:].reshape(1, D)
    cn = b_hh[2 * D:].reshape(1, D)

    return pl.pallas_call(
        _tc_body,
        out_shape=jax.ShapeDtypeStruct((F, D), jnp.float32),
    )(vpart, W_out0, W_out1, feature_emb,
      wrz, wn, un, brz, bn, cn,
      bias0.reshape(1, D), bias1.reshape(1, D))
